# Initial kernel scaffold; baseline (speedup 1.0000x reference)
import jax
import jax.numpy as jnp
from jax.experimental import pallas as pl


def kernel(hidden_states, gate_w, w1, w2, w3):
    raise NotImplementedError("write your pallas kernel here")


# speedup vs baseline: 1.5645x; 1.5645x over previous
import functools

import jax
import jax.numpy as jnp
from jax.experimental import pallas as pl
from jax.experimental.pallas import tpu as pltpu

TOPK = 2


def _moe_dense_body(x_ref, gate_ref, w1_ref, w3_ref, w2_ref,
                    out_ref, logits_ref, logits_s, nsteps_f):
    e = pl.program_id(0)
    f = pl.program_id(1)

    @pl.when(jnp.logical_and(e == 0, f == 0))
    def _():
        logits = jax.lax.dot_general(
            x_ref[...], gate_ref[...], (((1,), (1,)), ((), ())),
            preferred_element_type=jnp.float32)
        logits_ref[...] = logits
        logits_s[...] = logits

    logits = logits_s[...]
    p = jax.nn.softmax(logits, axis=1)
    m1 = jnp.max(p, axis=1, keepdims=True)
    p_wo_max = jnp.where(p >= m1, -jnp.inf, p)
    m2 = jnp.max(p_wo_max, axis=1, keepdims=True)
    emask = jax.lax.broadcasted_iota(jnp.int32, p.shape, 1) == e
    pe = jnp.sum(jnp.where(emask, p, 0.0), axis=1, keepdims=True)
    fw = jnp.where(pe >= m2, pe / (m1 + m2), 0.0)

    x = x_ref[...]
    h = jax.nn.silu(
        jnp.dot(x, w1_ref[0], preferred_element_type=jnp.float32)
    ) * jnp.dot(x, w3_ref[0], preferred_element_type=jnp.float32)
    y = jnp.dot(h, w2_ref[0], preferred_element_type=jnp.float32)

    @pl.when(jnp.logical_and(e == 0, f == 0))
    def _():
        out_ref[...] = fw * y

    @pl.when(jnp.logical_not(jnp.logical_and(e == 0, f == 0)))
    def _():
        out_ref[...] = out_ref[...] + fw * y


def kernel(hidden_states, gate_w, w1, w2, w3):
    batch, seq, hidden = hidden_states.shape
    T = batch * seq
    E, _, FFN = w1.shape
    x = hidden_states.reshape(T, hidden)

    BF = 1024
    NF = FFN // BF

    grid = (E, NF)
    out, logits = pl.pallas_call(
        functools.partial(_moe_dense_body, nsteps_f=NF),
        grid=grid,
        in_specs=[
            pl.BlockSpec((T, hidden), lambda e, f: (0, 0)),
            pl.BlockSpec((E, hidden), lambda e, f: (0, 0)),
            pl.BlockSpec((1, hidden, BF), lambda e, f: (e, 0, f)),
            pl.BlockSpec((1, hidden, BF), lambda e, f: (e, 0, f)),
            pl.BlockSpec((1, BF, hidden), lambda e, f: (e, f, 0)),
        ],
        out_specs=[
            pl.BlockSpec((T, hidden), lambda e, f: (0, 0)),
            pl.BlockSpec((T, E), lambda e, f: (0, 0)),
        ],
        out_shape=[
            jax.ShapeDtypeStruct((T, hidden), jnp.float32),
            jax.ShapeDtypeStruct((T, E), jnp.float32),
        ],
        scratch_shapes=[pltpu.VMEM((T, E), jnp.float32)],
        compiler_params=pltpu.CompilerParams(
            dimension_semantics=("arbitrary", "arbitrary")),
    )(x, gate_w, w1, w3, w2)

    return out.reshape(batch, seq, hidden), logits
